# Initial kernel scaffold; baseline (speedup 1.0000x reference)
import jax
import jax.numpy as jnp
from jax.experimental import pallas as pl


def kernel(x, edge_index, edge_attr, W_in, W_e1, Wself1, W1, b1, Wmsg2, W_e2, Wself2, b2, gate_w1, gate_b1, gate_w2, gate_b2):
    raise NotImplementedError("write your pallas kernel here")


# speedup vs baseline: 4.6829x; 4.6829x over previous
import functools

import jax
import jax.numpy as jnp
from jax import lax
from jax.experimental import pallas as pl
from jax.experimental.pallas import tpu as pltpu
from jax.experimental.pallas import tpu_sc as plsc

N = 10000
E = 160000
D_IN = 768
D_H = 256
D_OUT = 128
HALF = 128

CH = 128
NCHUNK = E // CH
NTILES = 16
ROWS_PER_TILE = 624
ROWS_TAIL = N - NTILES * ROWS_PER_TILE

NBIG1, NSMALL1, RBIG1 = 79, 78, 2
NBIG2, NSMALL2, RBIG2 = 40, 39, 1

def _dot(a, b):
    return jnp.dot(a.astype(jnp.bfloat16), b.astype(jnp.bfloat16),
                   preferred_element_type=jnp.float32)


def _bf16r(a):
    return a.astype(jnp.bfloat16).astype(jnp.float32)


def _proj_in_body(x_ref, w_ref, o_ref):
    res = _dot(x_ref[...], w_ref[...])
    o_ref[0] = res[:, :HALF]
    o_ref[1] = res[:, HALF:]


def _proj_in(x, W_in):
    R = 1000
    return pl.pallas_call(
        _proj_in_body,
        grid=(N // R,),
        in_specs=[
            pl.BlockSpec((R, D_IN), lambda i: (i, 0)),
            pl.BlockSpec((D_IN, D_H), lambda i: (0, 0)),
        ],
        out_specs=pl.BlockSpec((2, R, HALF), lambda i: (0, i, 0)),
        out_shape=jax.ShapeDtypeStruct((2, N, HALF), jnp.float32),
    )(x, W_in)


def _mid_body(a1a_ref, a1b_ref, ha_ref, hb_ref, w1_ref, ws1_ref, b1_ref,
              wm2_ref, ws2_ref, h1m_ref, h1s_ref):
    agg1 = jnp.concatenate([a1a_ref[...], a1b_ref[...]], axis=1)
    h = jnp.concatenate([ha_ref[...], hb_ref[...]], axis=1)
    pre = _dot(agg1, w1_ref[...]) + _dot(h, ws1_ref[...]) + b1_ref[...]
    h1 = jnp.maximum(pre, 0.0)
    h1m_ref[...] = _dot(h1, wm2_ref[...])
    h1s_ref[...] = _dot(h1, ws2_ref[...])


def _mid(agg1, h_flat, W1, Wself1, b1, Wmsg2, Wself2):
    R = 1000
    row = lambda i: (i, 0)
    rowb = lambda i: (N // R + i, 0)
    whole = lambda i: (0, 0)
    return pl.pallas_call(
        _mid_body,
        grid=(N // R,),
        in_specs=[
            pl.BlockSpec((R, HALF), row),
            pl.BlockSpec((R, HALF), rowb),
            pl.BlockSpec((R, HALF), row),
            pl.BlockSpec((R, HALF), rowb),
            pl.BlockSpec((D_H, D_H), whole),
            pl.BlockSpec((D_H, D_H), whole),
            pl.BlockSpec((1, D_H), whole),
            pl.BlockSpec((D_H, D_OUT), whole),
            pl.BlockSpec((D_H, D_OUT), whole),
        ],
        out_specs=[
            pl.BlockSpec((R, D_OUT), row),
            pl.BlockSpec((R, D_OUT), row),
        ],
        out_shape=[
            jax.ShapeDtypeStruct((N, D_OUT), jnp.float32),
            jax.ShapeDtypeStruct((N, D_OUT), jnp.float32),
        ],
    )(agg1, agg1, h_flat, h_flat, W1, Wself1, b1, Wmsg2, Wself2)


def _pool_body(a2_ref, h1s_ref, b2_ref, gw1_ref, gb1_ref,
               gw2_ref, gb2_ref, out_ref):
    a2 = a2_ref[...]
    ne = a2[:N] + a2[N:] + h1s_ref[...] + b2_ref[...]
    g1 = jnp.maximum(_dot(ne, gw1_ref[...]) + gb1_ref[...], 0.0)
    g1b = g1.astype(jnp.bfloat16).astype(jnp.float32)
    gate = jnp.sum(g1b * gw2_ref[...], axis=1, keepdims=True) + gb2_ref[0, 0]
    m = jnp.max(gate)
    ex = jnp.exp(gate - m)
    alpha = ex / jnp.sum(ex)
    out_ref[...] = jnp.sum(alpha * ne, axis=0, keepdims=True)


def _pool(a2, h1s, b2, gate_w1, gate_b1, gate_w2t, gate_b2):
    return pl.pallas_call(
        _pool_body,
        out_shape=jax.ShapeDtypeStruct((1, D_OUT), jnp.float32),
    )(a2, h1s, b2, gate_w1, gate_b1, gate_w2t, gate_b2)


def _load_weights(we_v, width):
    return [[we_v[pl.ds(k * width + j * 16, 16)] for j in range(width // 16)]
            for k in range(3)]


def _span(s, nbig, nsmall, r):
    n_s = jnp.where(s < r, nbig, nsmall)
    chunk0 = jnp.where(s < r, s * nbig, r * nbig + (s - r) * nsmall)
    return n_s, chunk0


def _edge_compute(rows_v, ea_v, wv, width):
    nreg = width // 16

    def group(g, _):
        off = g * 16
        a0v = ea_v[pl.ds(off, 16)]
        a1v = ea_v[pl.ds(CH + off, 16)]
        a2v = ea_v[pl.ds(2 * CH + off, 16)]
        for e in range(16):
            i = g * 16 + e
            a0 = a0v[e]
            a1 = a1v[e]
            a2 = a2v[e]
            for j in range(nreg):
                sl = pl.ds(j * 16, 16)
                v = (rows_v[i, sl]
                     + a0 * wv[0][j] + a1 * wv[1][j] + a2 * wv[2][j])
                rows_v[i, sl] = jnp.maximum(v, 0.0)
        return 0

    lax.fori_loop(0, CH // 16, group, 0)


def _zero_rows(rows_v, width):
    z = jnp.zeros((16,), jnp.float32)

    def body(i, _):
        for j in range(width // 16):
            rows_v[i, pl.ds(j * 16, 16)] = z
        return 0

    lax.fori_loop(0, CH, body, 0)


def _zero_table(rows_v, table_sh, tid):
    row0 = tid * ROWS_PER_TILE
    for k in range(4):
        pltpu.sync_copy(rows_v, table_sh.at[pl.ds(row0 + k * CH, CH)])
    pltpu.sync_copy(rows_v.at[pl.ds(0, 112)],
                    table_sh.at[pl.ds(row0 + 4 * CH, 112)])

    @pl.when(tid == 0)
    def _():
        pltpu.sync_copy(rows_v.at[pl.ds(0, ROWS_TAIL)],
                        table_sh.at[pl.ds(NTILES * ROWS_PER_TILE, ROWS_TAIL)])


def _copy_out(table_sh, out_hbm, tid, base):
    row0 = tid * ROWS_PER_TILE
    pltpu.sync_copy(table_sh.at[pl.ds(row0, ROWS_PER_TILE)],
                    out_hbm.at[pl.ds(base + row0, ROWS_PER_TILE)])

    @pl.when(tid == 0)
    def _():
        tb = NTILES * ROWS_PER_TILE
        pltpu.sync_copy(table_sh.at[pl.ds(tb, ROWS_TAIL)],
                        out_hbm.at[pl.ds(base + tb, ROWS_TAIL)])


def _pipeline(tbl_hbm, table_sh, src_hbm, dst_hbm, ea_hbm,
              sidx, didx, eab, rows, isem, gsem,
              wv, width, n_s, chunk0, nmax, idx_off):
    def issue_idx(p, k):
        g = chunk0 + k
        base = g * CH
        pltpu.async_copy(src_hbm.at[pl.ds(base, CH)], sidx[p], isem[p])
        pltpu.async_copy(dst_hbm.at[pl.ds(base, CH)], didx[p], isem[p])
        pltpu.async_copy(ea_hbm.at[pl.ds(g * 3 * CH, 3 * CH)], eab[p], isem[p])

    def wait_idx(p):
        pltpu.make_async_copy(src_hbm.at[pl.ds(0, CH)], sidx[p], isem[p]).wait()
        pltpu.make_async_copy(dst_hbm.at[pl.ds(0, CH)], didx[p], isem[p]).wait()
        pltpu.make_async_copy(ea_hbm.at[pl.ds(0, 3 * CH)], eab[p],
                              isem[p]).wait()

    def start_gather(p):
        for j in range(CH // 16):
            sl = pl.ds(j * 16, 16)
            sidx[p][sl] = sidx[p][sl] + idx_off
        pltpu.async_copy(tbl_hbm.at[sidx[p]], rows[p], gsem[p])

    def wait_gather(p):
        pltpu.make_async_copy(tbl_hbm.at[sidx[p]], rows[p], gsem[p]).wait()

    issue_idx(0, 0)
    wait_idx(0)
    start_gather(0)
    issue_idx(1, 1)

    def step(pa, pb, k):
        @pl.when(k + 1 < n_s)
        def _():
            wait_idx(pb)
            start_gather(pb)

        wait_gather(pa)
        _edge_compute(rows[pa], eab[pa], wv, width)
        pltpu.sync_copy(rows[pa], table_sh.at[didx[pa]], add=True)

        @pl.when(k + 2 < n_s)
        def _():
            issue_idx(pa, k + 2)

    def it(k, _):
        @pl.when(k < n_s)
        def _():
            @pl.when(k % 2 == 0)
            def _():
                step(0, 1, k)

            @pl.when(k % 2 == 1)
            def _():
                step(1, 0, k)

        return 0

    lax.fori_loop(0, nmax, it, 0)


_SC_SCRATCH = lambda width: [
    pltpu.VMEM((CH,), jnp.int32),
    pltpu.VMEM((CH,), jnp.int32),
    pltpu.VMEM((CH,), jnp.int32),
    pltpu.VMEM((CH,), jnp.int32),
    pltpu.VMEM((3 * CH,), jnp.float32),
    pltpu.VMEM((3 * CH,), jnp.float32),
    pltpu.VMEM((CH, width), jnp.float32),
    pltpu.VMEM((CH, width), jnp.float32),
    pltpu.VMEM((3 * width,), jnp.float32),
    pltpu.VMEM_SHARED((N, width), jnp.float32),
    pltpu.SemaphoreType.DMA,
    pltpu.SemaphoreType.DMA,
    pltpu.SemaphoreType.DMA,
    pltpu.SemaphoreType.DMA,
]


def _sc_layer1_body(h_hbm, src_hbm, dst_hbm, ea_hbm, we1_hbm, out_hbm,
                    sidx0, sidx1, didx0, didx1, ea0, ea1, rows0, rows1,
                    we1_v, table_sh, i0, i1, g0, g1):
    c = lax.axis_index("c")
    s = lax.axis_index("s")

    pltpu.sync_copy(we1_hbm.at[pl.ds(c * 3 * HALF, 3 * HALF)], we1_v)
    n_s, chunk0 = _span(s, NBIG1, NSMALL1, RBIG1)

    _zero_rows(rows0, HALF)
    _zero_table(rows0, table_sh, s)
    plsc.subcore_barrier()

    wv = _load_weights(we1_v, HALF)
    _pipeline(h_hbm, table_sh, src_hbm, dst_hbm, ea_hbm,
              (sidx0, sidx1), (didx0, didx1), (ea0, ea1), (rows0, rows1),
              (i0, i1), (g0, g1),
              wv, HALF, n_s, chunk0, NBIG1, c * N)

    plsc.subcore_barrier()
    _copy_out(table_sh, out_hbm, s, c * N)


def _sc_layer1(h_flat, src, dst, ea_flat, we1_flat):
    mesh = plsc.VectorSubcoreMesh(core_axis_name="c", subcore_axis_name="s")
    kern = functools.partial(
        pl.kernel,
        out_type=jax.ShapeDtypeStruct((2 * N, HALF), jnp.float32),
        mesh=mesh,
        scratch_types=_SC_SCRATCH(HALF),
    )(_sc_layer1_body)
    return kern(h_flat, src, dst, ea_flat, we1_flat)


def _sc_layer2_body(hm_hbm, src_hbm, dst_hbm, ea_hbm, we2_hbm, out_hbm,
                    sidx0, sidx1, didx0, didx1, ea0, ea1, rows0, rows1,
                    we2_v, table_sh, i0, i1, g0, g1):
    c = lax.axis_index("c")
    s = lax.axis_index("s")

    pltpu.sync_copy(we2_hbm, we2_v)
    n_s, chunk0 = _span(s, NBIG2, NSMALL2, RBIG2)
    chunk0 = c * (NCHUNK // 2) + chunk0

    _zero_rows(rows0, D_OUT)
    _zero_table(rows0, table_sh, s)
    plsc.subcore_barrier()

    wv = _load_weights(we2_v, D_OUT)
    _pipeline(hm_hbm, table_sh, src_hbm, dst_hbm, ea_hbm,
              (sidx0, sidx1), (didx0, didx1), (ea0, ea1), (rows0, rows1),
              (i0, i1), (g0, g1),
              wv, D_OUT, n_s, chunk0, NBIG2, 0)

    plsc.subcore_barrier()
    _copy_out(table_sh, out_hbm, s, c * N)


def _sc_layer2(h1m, src, dst, ea_flat, we2_flat):
    mesh = plsc.VectorSubcoreMesh(core_axis_name="c", subcore_axis_name="s")
    kern = functools.partial(
        pl.kernel,
        out_type=jax.ShapeDtypeStruct((2 * N, D_OUT), jnp.float32),
        mesh=mesh,
        scratch_types=_SC_SCRATCH(D_OUT),
    )(_sc_layer2_body)
    return kern(h1m, src, dst, ea_flat, we2_flat)


def kernel(x, edge_index, edge_attr, W_in, W_e1, Wself1, W1, b1, Wmsg2, W_e2,
           Wself2, b2, gate_w1, gate_b1, gate_w2, gate_b2):
    src = edge_index[0].astype(jnp.int32)
    dst = edge_index[1].astype(jnp.int32)

    h2 = _proj_in(x, W_in)
    h_flat = h2.reshape(2 * N, HALF)

    we1_flat = _bf16r(W_e1).reshape(3, 2, HALF).transpose(1, 0, 2).reshape(-1)
    ea_flat = _bf16r(edge_attr).reshape(NCHUNK, CH, 3)
    ea_flat = ea_flat.transpose(0, 2, 1).reshape(-1)

    agg1 = _sc_layer1(h_flat, src, dst, ea_flat, we1_flat)

    h1m, h1s = _mid(agg1, h_flat, W1, Wself1, b1.reshape(1, D_H),
                    Wmsg2, Wself2)

    a2 = _sc_layer2(h1m, src, dst, ea_flat, _bf16r(W_e2).reshape(-1))

    pooled = _pool(a2, h1s, b2.reshape(1, D_OUT),
                   gate_w1, gate_b1.reshape(1, D_H),
                   _bf16r(gate_w2).reshape(1, D_H), gate_b2.reshape(1, 1))
    return pooled


# speedup vs baseline: 4.8468x; 1.0350x over previous
import functools

import jax
import jax.numpy as jnp
from jax import lax
from jax.experimental import pallas as pl
from jax.experimental.pallas import tpu as pltpu
from jax.experimental.pallas import tpu_sc as plsc

N = 10000
E = 160000
D_IN = 768
D_H = 256
D_OUT = 128
HALF = 128

CH = 80
NCHUNK = E // CH
NTILES = 16
ROWS_PER_TILE = 624
ROWS_TAIL = N - NTILES * ROWS_PER_TILE

NBIG1, NSMALL1, RBIG1 = 125, 125, 0
NBIG2, NSMALL2, RBIG2 = 63, 62, 8

def _dot(a, b):
    return jnp.dot(a.astype(jnp.bfloat16), b.astype(jnp.bfloat16),
                   preferred_element_type=jnp.float32)


def _bf16r(a):
    return a.astype(jnp.bfloat16).astype(jnp.float32)


def _proj_in_body(x_ref, w_ref, o_ref):
    res = _dot(x_ref[...], w_ref[...])
    o_ref[0] = res[:, :HALF]
    o_ref[1] = res[:, HALF:]


def _proj_in(x, W_in):
    R = 1000
    return pl.pallas_call(
        _proj_in_body,
        grid=(N // R,),
        in_specs=[
            pl.BlockSpec((R, D_IN), lambda i: (i, 0)),
            pl.BlockSpec((D_IN, D_H), lambda i: (0, 0)),
        ],
        out_specs=pl.BlockSpec((2, R, HALF), lambda i: (0, i, 0)),
        out_shape=jax.ShapeDtypeStruct((2, N, HALF), jnp.float32),
    )(x, W_in)


def _mid_body(a1a_ref, a1b_ref, ha_ref, hb_ref, w1_ref, ws1_ref, b1_ref,
              wm2_ref, ws2_ref, h1m_ref, h1s_ref):
    agg1 = jnp.concatenate([a1a_ref[...], a1b_ref[...]], axis=1)
    h = jnp.concatenate([ha_ref[...], hb_ref[...]], axis=1)
    pre = _dot(agg1, w1_ref[...]) + _dot(h, ws1_ref[...]) + b1_ref[...]
    h1 = jnp.maximum(pre, 0.0)
    h1m_ref[...] = _dot(h1, wm2_ref[...])
    h1s_ref[...] = _dot(h1, ws2_ref[...])


def _mid(agg1, h_flat, W1, Wself1, b1, Wmsg2, Wself2):
    R = 1000
    row = lambda i: (i, 0)
    rowb = lambda i: (N // R + i, 0)
    whole = lambda i: (0, 0)
    return pl.pallas_call(
        _mid_body,
        grid=(N // R,),
        in_specs=[
            pl.BlockSpec((R, HALF), row),
            pl.BlockSpec((R, HALF), rowb),
            pl.BlockSpec((R, HALF), row),
            pl.BlockSpec((R, HALF), rowb),
            pl.BlockSpec((D_H, D_H), whole),
            pl.BlockSpec((D_H, D_H), whole),
            pl.BlockSpec((1, D_H), whole),
            pl.BlockSpec((D_H, D_OUT), whole),
            pl.BlockSpec((D_H, D_OUT), whole),
        ],
        out_specs=[
            pl.BlockSpec((R, D_OUT), row),
            pl.BlockSpec((R, D_OUT), row),
        ],
        out_shape=[
            jax.ShapeDtypeStruct((N, D_OUT), jnp.float32),
            jax.ShapeDtypeStruct((N, D_OUT), jnp.float32),
        ],
    )(agg1, agg1, h_flat, h_flat, W1, Wself1, b1, Wmsg2, Wself2)


def _pool_body(a2_ref, h1s_ref, b2_ref, gw1_ref, gb1_ref,
               gw2_ref, gb2_ref, out_ref):
    a2 = a2_ref[...]
    ne = a2[:N] + a2[N:] + h1s_ref[...] + b2_ref[...]
    g1 = jnp.maximum(_dot(ne, gw1_ref[...]) + gb1_ref[...], 0.0)
    g1b = g1.astype(jnp.bfloat16).astype(jnp.float32)
    gate = jnp.sum(g1b * gw2_ref[...], axis=1, keepdims=True) + gb2_ref[0, 0]
    m = jnp.max(gate)
    ex = jnp.exp(gate - m)
    alpha = ex / jnp.sum(ex)
    out_ref[...] = jnp.sum(alpha * ne, axis=0, keepdims=True)


def _pool(a2, h1s, b2, gate_w1, gate_b1, gate_w2t, gate_b2):
    return pl.pallas_call(
        _pool_body,
        out_shape=jax.ShapeDtypeStruct((1, D_OUT), jnp.float32),
    )(a2, h1s, b2, gate_w1, gate_b1, gate_w2t, gate_b2)


def _load_weights(we_v, width):
    return [[we_v[pl.ds(k * width + j * 16, 16)] for j in range(width // 16)]
            for k in range(3)]


def _span(s, nbig, nsmall, r):
    n_s = jnp.where(s < r, nbig, nsmall)
    chunk0 = jnp.where(s < r, s * nbig, r * nbig + (s - r) * nsmall)
    return n_s, chunk0


def _edge_compute(rows_v, ea_v, wv, width):
    nreg = width // 16

    def group(g, _):
        off = g * 16
        a0v = ea_v[pl.ds(off, 16)]
        a1v = ea_v[pl.ds(CH + off, 16)]
        a2v = ea_v[pl.ds(2 * CH + off, 16)]
        for e in range(16):
            i = g * 16 + e
            a0 = a0v[e]
            a1 = a1v[e]
            a2 = a2v[e]
            for j in range(nreg):
                sl = pl.ds(j * 16, 16)
                v = (rows_v[i, sl]
                     + a0 * wv[0][j] + a1 * wv[1][j] + a2 * wv[2][j])
                rows_v[i, sl] = jnp.maximum(v, 0.0)
        return 0

    lax.fori_loop(0, CH // 16, group, 0)


def _zero_rows(rows_v, width):
    z = jnp.zeros((16,), jnp.float32)

    def body(i, _):
        for j in range(width // 16):
            rows_v[i, pl.ds(j * 16, 16)] = z
        return 0

    lax.fori_loop(0, CH, body, 0)


def _zero_table(rows_v, table_sh, tid):
    row0 = tid * ROWS_PER_TILE
    for k in range(7):
        pltpu.sync_copy(rows_v, table_sh.at[pl.ds(row0 + k * CH, CH)])
    pltpu.sync_copy(rows_v.at[pl.ds(0, 64)],
                    table_sh.at[pl.ds(row0 + 7 * CH, 64)])

    @pl.when(tid == 0)
    def _():
        pltpu.sync_copy(rows_v.at[pl.ds(0, ROWS_TAIL)],
                        table_sh.at[pl.ds(NTILES * ROWS_PER_TILE, ROWS_TAIL)])


def _copy_out(table_sh, out_hbm, tid, base):
    row0 = tid * ROWS_PER_TILE
    pltpu.sync_copy(table_sh.at[pl.ds(row0, ROWS_PER_TILE)],
                    out_hbm.at[pl.ds(base + row0, ROWS_PER_TILE)])

    @pl.when(tid == 0)
    def _():
        tb = NTILES * ROWS_PER_TILE
        pltpu.sync_copy(table_sh.at[pl.ds(tb, ROWS_TAIL)],
                        out_hbm.at[pl.ds(base + tb, ROWS_TAIL)])


def _pipeline(tbl_hbm, table_sh, src_hbm, dst_hbm, ea_hbm,
              sidx, didx, eab, rows, isem, dsem, gsem, ssem,
              wv, width, n_s, chunk0, nmax, idx_off):
    def issue_se(p, k):
        g = chunk0 + k
        pltpu.async_copy(src_hbm.at[pl.ds(g * CH, CH)], sidx[p], isem[p])
        pltpu.async_copy(ea_hbm.at[pl.ds(g * 3 * CH, 3 * CH)], eab[p], isem[p])

    def wait_se(p):
        pltpu.make_async_copy(src_hbm.at[pl.ds(0, CH)], sidx[p], isem[p]).wait()
        pltpu.make_async_copy(ea_hbm.at[pl.ds(0, 3 * CH)], eab[p],
                              isem[p]).wait()

    def issue_d(p, k):
        g = chunk0 + k
        pltpu.async_copy(dst_hbm.at[pl.ds(g * CH, CH)], didx[p], dsem[p])

    def wait_d(p):
        pltpu.make_async_copy(dst_hbm.at[pl.ds(0, CH)], didx[p], dsem[p]).wait()

    def start_gather(p):
        for j in range(CH // 16):
            sl = pl.ds(j * 16, 16)
            sidx[p][sl] = sidx[p][sl] + idx_off
        pltpu.async_copy(tbl_hbm.at[sidx[p]], rows[p], gsem[p])

    def wait_gather(p):
        pltpu.make_async_copy(tbl_hbm.at[sidx[p]], rows[p], gsem[p]).wait()

    def wait_scatter(p):
        pltpu.make_async_copy(rows[p], table_sh.at[didx[p]], ssem[p]).wait()

    issue_se(0, 0)
    issue_d(0, 0)
    wait_se(0)
    start_gather(0)
    issue_se(1, 1)

    def step(pa, pb, pc, k):
        @pl.when(k + 1 < n_s)
        def _():
            wait_se(pb)

            @pl.when(k >= 2)
            def _():
                wait_scatter(pb)

            issue_d(pb, k + 1)
            start_gather(pb)

        wait_gather(pa)
        _edge_compute(rows[pa], eab[pa], wv, width)
        wait_d(pa)
        pltpu.async_copy(rows[pa], table_sh.at[didx[pa]], ssem[pa], add=True)

        @pl.when(k + 2 < n_s)
        def _():
            issue_se(pc, k + 2)

    def it(k, _):
        @pl.when(k < n_s)
        def _():
            for q in range(3):
                @pl.when(k % 3 == q)
                def _(q=q):
                    step(q, (q + 1) % 3, (q + 2) % 3, k)

        return 0

    lax.fori_loop(0, nmax, it, 0)
    for p in range(3):
        wait_scatter(p)


_SC_SCRATCH = lambda width: (
    [pltpu.VMEM((CH,), jnp.int32)] * 3
    + [pltpu.VMEM((CH,), jnp.int32)] * 3
    + [pltpu.VMEM((3 * CH,), jnp.float32)] * 3
    + [pltpu.VMEM((CH, width), jnp.float32)] * 3
    + [pltpu.VMEM((3 * width,), jnp.float32)]
    + [pltpu.VMEM_SHARED((N, width), jnp.float32)]
    + [pltpu.SemaphoreType.DMA] * 12
)


def _sc_layer1_body(h_hbm, src_hbm, dst_hbm, ea_hbm, we1_hbm, out_hbm,
                    sidx0, sidx1, sidx2, didx0, didx1, didx2,
                    ea0, ea1, ea2, rows0, rows1, rows2,
                    we1_v, table_sh,
                    i0, i1, i2, d0, d1, d2, g0, g1, g2, s0, s1, s2):
    c = lax.axis_index("c")
    s = lax.axis_index("s")

    pltpu.sync_copy(we1_hbm.at[pl.ds(c * 3 * HALF, 3 * HALF)], we1_v)
    n_s, chunk0 = _span(s, NBIG1, NSMALL1, RBIG1)

    _zero_rows(rows0, HALF)
    _zero_table(rows0, table_sh, s)
    plsc.subcore_barrier()

    wv = _load_weights(we1_v, HALF)
    _pipeline(h_hbm, table_sh, src_hbm, dst_hbm, ea_hbm,
              (sidx0, sidx1, sidx2), (didx0, didx1, didx2),
              (ea0, ea1, ea2), (rows0, rows1, rows2),
              (i0, i1, i2), (d0, d1, d2), (g0, g1, g2), (s0, s1, s2),
              wv, HALF, n_s, chunk0, NBIG1, c * N)

    plsc.subcore_barrier()
    _copy_out(table_sh, out_hbm, s, c * N)


def _sc_layer1(h_flat, src, dst, ea_flat, we1_flat):
    mesh = plsc.VectorSubcoreMesh(core_axis_name="c", subcore_axis_name="s")
    kern = functools.partial(
        pl.kernel,
        out_type=jax.ShapeDtypeStruct((2 * N, HALF), jnp.float32),
        mesh=mesh,
        scratch_types=_SC_SCRATCH(HALF),
    )(_sc_layer1_body)
    return kern(h_flat, src, dst, ea_flat, we1_flat)


def _sc_layer2_body(hm_hbm, src_hbm, dst_hbm, ea_hbm, we2_hbm, out_hbm,
                    sidx0, sidx1, sidx2, didx0, didx1, didx2,
                    ea0, ea1, ea2, rows0, rows1, rows2,
                    we2_v, table_sh,
                    i0, i1, i2, d0, d1, d2, g0, g1, g2, s0, s1, s2):
    c = lax.axis_index("c")
    s = lax.axis_index("s")

    pltpu.sync_copy(we2_hbm, we2_v)
    n_s, chunk0 = _span(s, NBIG2, NSMALL2, RBIG2)
    chunk0 = c * (NCHUNK // 2) + chunk0

    _zero_rows(rows0, D_OUT)
    _zero_table(rows0, table_sh, s)
    plsc.subcore_barrier()

    wv = _load_weights(we2_v, D_OUT)
    _pipeline(hm_hbm, table_sh, src_hbm, dst_hbm, ea_hbm,
              (sidx0, sidx1, sidx2), (didx0, didx1, didx2),
              (ea0, ea1, ea2), (rows0, rows1, rows2),
              (i0, i1, i2), (d0, d1, d2), (g0, g1, g2), (s0, s1, s2),
              wv, D_OUT, n_s, chunk0, NBIG2, 0)

    plsc.subcore_barrier()
    _copy_out(table_sh, out_hbm, s, c * N)


def _sc_layer2(h1m, src, dst, ea_flat, we2_flat):
    mesh = plsc.VectorSubcoreMesh(core_axis_name="c", subcore_axis_name="s")
    kern = functools.partial(
        pl.kernel,
        out_type=jax.ShapeDtypeStruct((2 * N, D_OUT), jnp.float32),
        mesh=mesh,
        scratch_types=_SC_SCRATCH(D_OUT),
    )(_sc_layer2_body)
    return kern(h1m, src, dst, ea_flat, we2_flat)


def kernel(x, edge_index, edge_attr, W_in, W_e1, Wself1, W1, b1, Wmsg2, W_e2,
           Wself2, b2, gate_w1, gate_b1, gate_w2, gate_b2):
    src = edge_index[0].astype(jnp.int32)
    dst = edge_index[1].astype(jnp.int32)

    h2 = _proj_in(x, W_in)
    h_flat = h2.reshape(2 * N, HALF)

    we1_flat = _bf16r(W_e1).reshape(3, 2, HALF).transpose(1, 0, 2).reshape(-1)
    ea_flat = _bf16r(edge_attr).reshape(NCHUNK, CH, 3)
    ea_flat = ea_flat.transpose(0, 2, 1).reshape(-1)

    agg1 = _sc_layer1(h_flat, src, dst, ea_flat, we1_flat)

    h1m, h1s = _mid(agg1, h_flat, W1, Wself1, b1.reshape(1, D_H),
                    Wmsg2, Wself2)

    a2 = _sc_layer2(h1m, src, dst, ea_flat, _bf16r(W_e2).reshape(-1))

    pooled = _pool(a2, h1s, b2.reshape(1, D_OUT),
                   gate_w1, gate_b1.reshape(1, D_H),
                   _bf16r(gate_w2).reshape(1, D_H), gate_b2.reshape(1, 1))
    return pooled


# speedup vs baseline: 5.2333x; 1.0797x over previous
import functools

import jax
import jax.numpy as jnp
from jax import lax
from jax.experimental import pallas as pl
from jax.experimental.pallas import tpu as pltpu
from jax.experimental.pallas import tpu_sc as plsc

N = 10000
E = 160000
D_IN = 768
D_H = 256
D_OUT = 128
HALF = 128

CH = 80
NCHUNK = E // CH
NTILES = 16
ROWS_PER_TILE = 624
ROWS_TAIL = N - NTILES * ROWS_PER_TILE

NBIG1, NSMALL1, RBIG1 = 125, 125, 0
NBIG2, NSMALL2, RBIG2 = 63, 62, 8

def _dot(a, b):
    return jnp.dot(a.astype(jnp.bfloat16), b.astype(jnp.bfloat16),
                   preferred_element_type=jnp.float32)


def _bf16r(a):
    return a.astype(jnp.bfloat16).astype(jnp.float32)


def _proj_in_body(x_ref, w_ref, o_ref):
    res = _dot(x_ref[...], w_ref[...])
    o_ref[0] = res[:, :HALF]
    o_ref[1] = res[:, HALF:]


def _proj_in(x, W_in):
    R = 1000
    return pl.pallas_call(
        _proj_in_body,
        grid=(N // R,),
        in_specs=[
            pl.BlockSpec((R, D_IN), lambda i: (i, 0)),
            pl.BlockSpec((D_IN, D_H), lambda i: (0, 0)),
        ],
        out_specs=pl.BlockSpec((2, R, HALF), lambda i: (0, i, 0)),
        out_shape=jax.ShapeDtypeStruct((2, N, HALF), jnp.float32),
    )(x, W_in)


def _mid_body(a1a_ref, a1b_ref, ha_ref, hb_ref, w1_ref, ws1_ref, b1_ref,
              wm2_ref, ws2_ref, h1m_ref, h1s_ref):
    agg1 = jnp.concatenate([a1a_ref[...], a1b_ref[...]], axis=1)
    h = jnp.concatenate([ha_ref[...], hb_ref[...]], axis=1)
    pre = _dot(agg1, w1_ref[...]) + _dot(h, ws1_ref[...]) + b1_ref[...]
    h1 = jnp.maximum(pre, 0.0)
    h1m_ref[...] = _dot(h1, wm2_ref[...])
    h1s_ref[...] = _dot(h1, ws2_ref[...])


def _mid(agg1, h_flat, W1, Wself1, b1, Wmsg2, Wself2):
    R = 1000
    row = lambda i: (i, 0)
    rowb = lambda i: (N // R + i, 0)
    whole = lambda i: (0, 0)
    return pl.pallas_call(
        _mid_body,
        grid=(N // R,),
        in_specs=[
            pl.BlockSpec((R, HALF), row),
            pl.BlockSpec((R, HALF), rowb),
            pl.BlockSpec((R, HALF), row),
            pl.BlockSpec((R, HALF), rowb),
            pl.BlockSpec((D_H, D_H), whole),
            pl.BlockSpec((D_H, D_H), whole),
            pl.BlockSpec((1, D_H), whole),
            pl.BlockSpec((D_H, D_OUT), whole),
            pl.BlockSpec((D_H, D_OUT), whole),
        ],
        out_specs=[
            pl.BlockSpec((R, D_OUT), row),
            pl.BlockSpec((R, D_OUT), row),
        ],
        out_shape=[
            jax.ShapeDtypeStruct((N, D_OUT), jnp.float32),
            jax.ShapeDtypeStruct((N, D_OUT), jnp.float32),
        ],
    )(agg1, agg1, h_flat, h_flat, W1, Wself1, b1, Wmsg2, Wself2)


def _pool_body(a2_ref, h1s_ref, b2_ref, gw1_ref, gb1_ref,
               gw2_ref, gb2_ref, out_ref):
    a2 = a2_ref[...]
    ne = a2[:N] + a2[N:] + h1s_ref[...] + b2_ref[...]
    g1 = jnp.maximum(_dot(ne, gw1_ref[...]) + gb1_ref[...], 0.0)
    g1b = g1.astype(jnp.bfloat16).astype(jnp.float32)
    gate = jnp.sum(g1b * gw2_ref[...], axis=1, keepdims=True) + gb2_ref[0, 0]
    m = jnp.max(gate)
    ex = jnp.exp(gate - m)
    alpha = ex / jnp.sum(ex)
    out_ref[...] = jnp.sum(alpha * ne, axis=0, keepdims=True)


def _pool(a2, h1s, b2, gate_w1, gate_b1, gate_w2t, gate_b2):
    return pl.pallas_call(
        _pool_body,
        out_shape=jax.ShapeDtypeStruct((1, D_OUT), jnp.float32),
    )(a2, h1s, b2, gate_w1, gate_b1, gate_w2t, gate_b2)


def _load_weights(we_v, width):
    return [[we_v[pl.ds(k * width + j * 16, 16)] for j in range(width // 16)]
            for k in range(3)]


def _span(s, nbig, nsmall, r):
    n_s = jnp.where(s < r, nbig, nsmall)
    chunk0 = jnp.where(s < r, s * nbig, r * nbig + (s - r) * nsmall)
    return n_s, chunk0


def _edge_compute(rows_v, ea_v, wv, width):
    nreg = width // 16

    @plsc.parallel_loop(0, CH // 16)
    def group(g):
        off = g * 16
        a0v = ea_v[pl.ds(off, 16)]
        a1v = ea_v[pl.ds(CH + off, 16)]
        a2v = ea_v[pl.ds(2 * CH + off, 16)]
        for e in range(16):
            i = g * 16 + e
            a0 = a0v[e]
            a1 = a1v[e]
            a2 = a2v[e]
            for j in range(nreg):
                sl = pl.ds(j * 16, 16)
                v = (rows_v[i, sl]
                     + a0 * wv[0][j] + a1 * wv[1][j] + a2 * wv[2][j])
                rows_v[i, sl] = jnp.maximum(v, 0.0)


def _zero_rows(rows_v, width):
    z = jnp.zeros((16,), jnp.float32)

    def body(i, _):
        for j in range(width // 16):
            rows_v[i, pl.ds(j * 16, 16)] = z
        return 0

    lax.fori_loop(0, CH, body, 0)


def _zero_table(rows_v, table_sh, tid):
    row0 = tid * ROWS_PER_TILE
    for k in range(7):
        pltpu.sync_copy(rows_v, table_sh.at[pl.ds(row0 + k * CH, CH)])
    pltpu.sync_copy(rows_v.at[pl.ds(0, 64)],
                    table_sh.at[pl.ds(row0 + 7 * CH, 64)])

    @pl.when(tid == 0)
    def _():
        pltpu.sync_copy(rows_v.at[pl.ds(0, ROWS_TAIL)],
                        table_sh.at[pl.ds(NTILES * ROWS_PER_TILE, ROWS_TAIL)])


def _copy_out(table_sh, out_hbm, tid, base):
    row0 = tid * ROWS_PER_TILE
    pltpu.sync_copy(table_sh.at[pl.ds(row0, ROWS_PER_TILE)],
                    out_hbm.at[pl.ds(base + row0, ROWS_PER_TILE)])

    @pl.when(tid == 0)
    def _():
        tb = NTILES * ROWS_PER_TILE
        pltpu.sync_copy(table_sh.at[pl.ds(tb, ROWS_TAIL)],
                        out_hbm.at[pl.ds(base + tb, ROWS_TAIL)])


def _pipeline(tbl_hbm, table_sh, src_hbm, dst_hbm, ea_hbm,
              sidx, didx, eab, rows, isem, dsem, gsem, ssem,
              wv, width, n_s, chunk0, nmax, idx_off):
    def issue_se(p, k):
        g = chunk0 + k
        pltpu.async_copy(src_hbm.at[pl.ds(g * CH, CH)], sidx[p], isem[p])
        pltpu.async_copy(ea_hbm.at[pl.ds(g * 3 * CH, 3 * CH)], eab[p], isem[p])

    def wait_se(p):
        pltpu.make_async_copy(src_hbm.at[pl.ds(0, CH)], sidx[p], isem[p]).wait()
        pltpu.make_async_copy(ea_hbm.at[pl.ds(0, 3 * CH)], eab[p],
                              isem[p]).wait()

    def issue_d(p, k):
        g = chunk0 + k
        pltpu.async_copy(dst_hbm.at[pl.ds(g * CH, CH)], didx[p], dsem[p])

    def wait_d(p):
        pltpu.make_async_copy(dst_hbm.at[pl.ds(0, CH)], didx[p], dsem[p]).wait()

    def start_gather(p):
        for j in range(CH // 16):
            sl = pl.ds(j * 16, 16)
            sidx[p][sl] = sidx[p][sl] + idx_off
        pltpu.async_copy(tbl_hbm.at[sidx[p]], rows[p], gsem[p])

    def wait_gather(p):
        pltpu.make_async_copy(tbl_hbm.at[sidx[p]], rows[p], gsem[p]).wait()

    def wait_scatter(p):
        pltpu.make_async_copy(rows[p], table_sh.at[didx[p]], ssem[p]).wait()

    issue_se(0, 0)
    issue_d(0, 0)
    wait_se(0)
    start_gather(0)
    issue_se(1, 1)

    def step(pa, pb, pc, k):
        @pl.when(k + 1 < n_s)
        def _():
            wait_se(pb)

            @pl.when(k >= 2)
            def _():
                wait_scatter(pb)

            issue_d(pb, k + 1)
            start_gather(pb)

        wait_gather(pa)
        _edge_compute(rows[pa], eab[pa], wv, width)
        wait_d(pa)
        pltpu.async_copy(rows[pa], table_sh.at[didx[pa]], ssem[pa], add=True)

        @pl.when(k + 2 < n_s)
        def _():
            issue_se(pc, k + 2)

    def it(k, _):
        @pl.when(k < n_s)
        def _():
            for q in range(3):
                @pl.when(k % 3 == q)
                def _(q=q):
                    step(q, (q + 1) % 3, (q + 2) % 3, k)

        return 0

    lax.fori_loop(0, nmax, it, 0)
    for p in range(3):
        wait_scatter(p)


_SC_SCRATCH = lambda width: (
    [pltpu.VMEM((CH,), jnp.int32)] * 3
    + [pltpu.VMEM((CH,), jnp.int32)] * 3
    + [pltpu.VMEM((3 * CH,), jnp.float32)] * 3
    + [pltpu.VMEM((CH, width), jnp.float32)] * 3
    + [pltpu.VMEM((3 * width,), jnp.float32)]
    + [pltpu.VMEM_SHARED((N, width), jnp.float32)]
    + [pltpu.SemaphoreType.DMA] * 12
)


def _sc_layer1_body(h_hbm, src_hbm, dst_hbm, ea_hbm, we1_hbm, out_hbm,
                    sidx0, sidx1, sidx2, didx0, didx1, didx2,
                    ea0, ea1, ea2, rows0, rows1, rows2,
                    we1_v, table_sh,
                    i0, i1, i2, d0, d1, d2, g0, g1, g2, s0, s1, s2):
    c = lax.axis_index("c")
    s = lax.axis_index("s")

    pltpu.sync_copy(we1_hbm.at[pl.ds(c * 3 * HALF, 3 * HALF)], we1_v)
    n_s, chunk0 = _span(s, NBIG1, NSMALL1, RBIG1)

    _zero_rows(rows0, HALF)
    _zero_table(rows0, table_sh, s)
    plsc.subcore_barrier()

    wv = _load_weights(we1_v, HALF)
    _pipeline(h_hbm, table_sh, src_hbm, dst_hbm, ea_hbm,
              (sidx0, sidx1, sidx2), (didx0, didx1, didx2),
              (ea0, ea1, ea2), (rows0, rows1, rows2),
              (i0, i1, i2), (d0, d1, d2), (g0, g1, g2), (s0, s1, s2),
              wv, HALF, n_s, chunk0, NBIG1, c * N)

    plsc.subcore_barrier()
    _copy_out(table_sh, out_hbm, s, c * N)


def _sc_layer1(h_flat, src, dst, ea_flat, we1_flat):
    mesh = plsc.VectorSubcoreMesh(core_axis_name="c", subcore_axis_name="s")
    kern = functools.partial(
        pl.kernel,
        out_type=jax.ShapeDtypeStruct((2 * N, HALF), jnp.float32),
        mesh=mesh,
        scratch_types=_SC_SCRATCH(HALF),
    )(_sc_layer1_body)
    return kern(h_flat, src, dst, ea_flat, we1_flat)


def _sc_layer2_body(hm_hbm, src_hbm, dst_hbm, ea_hbm, we2_hbm, out_hbm,
                    sidx0, sidx1, sidx2, didx0, didx1, didx2,
                    ea0, ea1, ea2, rows0, rows1, rows2,
                    we2_v, table_sh,
                    i0, i1, i2, d0, d1, d2, g0, g1, g2, s0, s1, s2):
    c = lax.axis_index("c")
    s = lax.axis_index("s")

    pltpu.sync_copy(we2_hbm, we2_v)
    n_s, chunk0 = _span(s, NBIG2, NSMALL2, RBIG2)
    chunk0 = c * (NCHUNK // 2) + chunk0

    _zero_rows(rows0, D_OUT)
    _zero_table(rows0, table_sh, s)
    plsc.subcore_barrier()

    wv = _load_weights(we2_v, D_OUT)
    _pipeline(hm_hbm, table_sh, src_hbm, dst_hbm, ea_hbm,
              (sidx0, sidx1, sidx2), (didx0, didx1, didx2),
              (ea0, ea1, ea2), (rows0, rows1, rows2),
              (i0, i1, i2), (d0, d1, d2), (g0, g1, g2), (s0, s1, s2),
              wv, D_OUT, n_s, chunk0, NBIG2, 0)

    plsc.subcore_barrier()
    _copy_out(table_sh, out_hbm, s, c * N)


def _sc_layer2(h1m, src, dst, ea_flat, we2_flat):
    mesh = plsc.VectorSubcoreMesh(core_axis_name="c", subcore_axis_name="s")
    kern = functools.partial(
        pl.kernel,
        out_type=jax.ShapeDtypeStruct((2 * N, D_OUT), jnp.float32),
        mesh=mesh,
        scratch_types=_SC_SCRATCH(D_OUT),
    )(_sc_layer2_body)
    return kern(h1m, src, dst, ea_flat, we2_flat)


def kernel(x, edge_index, edge_attr, W_in, W_e1, Wself1, W1, b1, Wmsg2, W_e2,
           Wself2, b2, gate_w1, gate_b1, gate_w2, gate_b2):
    src = edge_index[0].astype(jnp.int32)
    dst = edge_index[1].astype(jnp.int32)

    h2 = _proj_in(x, W_in)
    h_flat = h2.reshape(2 * N, HALF)

    we1_flat = _bf16r(W_e1).reshape(3, 2, HALF).transpose(1, 0, 2).reshape(-1)
    ea_flat = _bf16r(edge_attr).reshape(NCHUNK, CH, 3)
    ea_flat = ea_flat.transpose(0, 2, 1).reshape(-1)

    agg1 = _sc_layer1(h_flat, src, dst, ea_flat, we1_flat)

    h1m, h1s = _mid(agg1, h_flat, W1, Wself1, b1.reshape(1, D_H),
                    Wmsg2, Wself2)

    a2 = _sc_layer2(h1m, src, dst, ea_flat, _bf16r(W_e2).reshape(-1))

    pooled = _pool(a2, h1s, b2.reshape(1, D_OUT),
                   gate_w1, gate_b1.reshape(1, D_H),
                   _bf16r(gate_w2).reshape(1, D_H), gate_b2.reshape(1, 1))
    return pooled
